# baseline (speedup 1.0000x reference)
import functools

import jax
import jax.numpy as jnp
from jax import lax
from jax.experimental import pallas as pl
from jax.experimental.pallas import tpu as pltpu
from jax.experimental.pallas import tpu_sc as plsc

H = 512
N_T = 10000
N_E = 10000
E = 160000
NUM_GRAPHS = 64
NUM_CLASSES = 32

NW = 32
EPW_P = 5120
K = 64
NCH = EPW_P // K
PAD_E = NW * EPW_P - E
N_PAD = 10240
RPT = N_PAD // 16
DUMP = 10200

R_BLK = 1000
N_BLK = N_T // R_BLK


def _sc_body(xt2, xe, sf0, sf1, sbt, dff, dbt, zrow, zcnt, ones_h,
             off0, off1, obt, ocf, ocb,
             acc, cntf, cntb, sidx, didx, rows0, rows1, ones_v,
             sem0, sem1):
  c = lax.axis_index("c")
  s = lax.axis_index("s")
  wid = s * 2 + c
  base = s * RPT

  def do_pass(x_hbm, src_hbm, dst_hbm, cnt_ref):
    pltpu.sync_copy(src_hbm.at[wid], sidx)
    pltpu.sync_copy(dst_hbm.at[wid], didx)
    dummy = x_hbm.at[pl.ds(0, K)]
    pltpu.async_copy(x_hbm.at[sidx.at[0]], rows0, sem0)

    def scat(buf, j):
      pltpu.sync_copy(buf, acc.at[didx.at[j]], add=True)
      if cnt_ref is not None:
        pltpu.sync_copy(ones_v, cnt_ref.at[didx.at[j]], add=True)

    def pair(g, carry):
      j0 = 2 * g
      pltpu.make_async_copy(dummy, rows0, sem0).wait()
      pltpu.async_copy(x_hbm.at[sidx.at[j0 + 1]], rows1, sem1)
      scat(rows0, j0)
      pltpu.make_async_copy(dummy, rows1, sem1).wait()
      nxt = jnp.minimum(j0 + 2, NCH - 1)
      pltpu.async_copy(x_hbm.at[sidx.at[nxt]], rows0, sem0)
      scat(rows1, j0 + 1)
      return carry

    lax.fori_loop(0, NCH // 2, pair, 0)
    pltpu.make_async_copy(dummy, rows0, sem0).wait()

  def zero_acc():
    pltpu.sync_copy(zrow.at[pl.ds(base, RPT)], acc.at[pl.ds(base, RPT)])

  zero_acc()
  pltpu.sync_copy(zcnt.at[pl.ds(base, RPT)], cntf.at[pl.ds(base, RPT)])
  pltpu.sync_copy(zcnt.at[pl.ds(base, RPT)], cntb.at[pl.ds(base, RPT)])
  pltpu.sync_copy(ones_h, ones_v)
  plsc.subcore_barrier()

  do_pass(xt2, sf0, dff, cntf)
  plsc.subcore_barrier()
  pltpu.sync_copy(acc.at[pl.ds(base, RPT)], off0.at[c, pl.ds(base, RPT)])
  zero_acc()
  plsc.subcore_barrier()

  do_pass(xt2, sf1, dff, None)
  plsc.subcore_barrier()
  pltpu.sync_copy(acc.at[pl.ds(base, RPT)], off1.at[c, pl.ds(base, RPT)])
  zero_acc()
  plsc.subcore_barrier()

  do_pass(xe, sbt, dbt, cntb)
  plsc.subcore_barrier()
  pltpu.sync_copy(acc.at[pl.ds(base, RPT)], obt.at[c, pl.ds(base, RPT)])
  pltpu.sync_copy(cntf.at[pl.ds(base, RPT)], ocf.at[c, pl.ds(base, RPT)])
  pltpu.sync_copy(cntb.at[pl.ds(base, RPT)], ocb.at[c, pl.ds(base, RPT)])


def _sc_aggregate(xt2, xe, sf0, sf1, sbt, dff, dbt, zrow, zcnt, ones_h):
  mesh = plsc.VectorSubcoreMesh(core_axis_name="c", subcore_axis_name="s")
  f32 = jnp.float32
  return pl.kernel(
      _sc_body,
      out_type=(
          jax.ShapeDtypeStruct((2, N_PAD, 128), f32),
          jax.ShapeDtypeStruct((2, N_PAD, 128), f32),
          jax.ShapeDtypeStruct((2, N_PAD, 128), f32),
          jax.ShapeDtypeStruct((2, N_PAD, 16), f32),
          jax.ShapeDtypeStruct((2, N_PAD, 16), f32),
      ),
      mesh=mesh,
      scratch_types=[
          pltpu.VMEM_SHARED((N_PAD, 128), f32),
          pltpu.VMEM_SHARED((N_PAD, 16), f32),
          pltpu.VMEM_SHARED((N_PAD, 16), f32),
          pltpu.VMEM((NCH, K), jnp.int32),
          pltpu.VMEM((NCH, K), jnp.int32),
          pltpu.VMEM((K, 128), f32),
          pltpu.VMEM((K, 128), f32),
          pltpu.VMEM((K, 16), f32),
          pltpu.SemaphoreType.DMA,
          pltpu.SemaphoreType.DMA,
      ],
      compiler_params=pltpu.CompilerParams(use_tc_tiling_on_sc=False),
  )(xt2, xe, sf0, sf1, sbt, dff, dbt, zrow, zcnt, ones_h)


def _tc_body(aff0, aff1, abt, cff, cbt, xt, tb,
             wpt, wpe, wlff, wlbt, wrff, wrbt, blff, blbt,
             wlin, blin, wh, bh,
             head_o,
             A_ff, A_bt, A_r, psum, pcnt):
  i = pl.program_id(0)
  f32 = jnp.float32
  dn = (((0,), (1,)), ((), ()))

  @pl.when(i == 0)
  def _():
    A_ff[...] = lax.dot_general(wpt[...], wlff[...], dn,
                                preferred_element_type=f32,
                                precision=lax.Precision.HIGHEST)
    A_bt[...] = lax.dot_general(wpe[...], wlbt[...], dn,
                                preferred_element_type=f32,
                                precision=lax.Precision.HIGHEST)
    A_r[...] = lax.dot_general(wpt[...], wrff[...] + wrbt[...], dn,
                               preferred_element_type=f32,
                                precision=lax.Precision.HIGHEST)
    psum[...] = jnp.zeros_like(psum)
    pcnt[...] = jnp.zeros_like(pcnt)

  nff = jnp.maximum(cff[0, :, 0] + cff[1, :, 0], 1.0)
  nbt = jnp.maximum(cbt[0, :, 0] + cbt[1, :, 0], 1.0)
  mff = jnp.concatenate([aff0[0] + aff0[1], aff1[0] + aff1[1]], axis=1)
  mff = mff / nff[:, None]
  mbt = (abt[0] + abt[1]) / nbt[:, None]

  dnm = (((1,), (0,)), ((), ()))
  o = (lax.dot_general(mff, A_ff[...], dnm, preferred_element_type=f32,
                                precision=lax.Precision.HIGHEST)
       + lax.dot_general(mbt, A_bt[...], dnm, preferred_element_type=f32,
                                precision=lax.Precision.HIGHEST)
       + lax.dot_general(xt[...], A_r[...], dnm, preferred_element_type=f32,
                                precision=lax.Precision.HIGHEST)
       + (blff[...] + blbt[...])[None, :])
  dnt = (((1,), (1,)), ((), ()))
  t = jnp.maximum(
      lax.dot_general(o, wlin[...], dnt, preferred_element_type=f32,
                                precision=lax.Precision.HIGHEST)
      + blin[...][None, :], 0.0)

  ids = tb[0, 0, :]
  p = (ids[:, None] == lax.broadcasted_iota(jnp.int32, (R_BLK, NUM_GRAPHS),
                                            1)).astype(f32)
  psum[...] += lax.dot_general(p, t, (((0,), (0,)), ((), ())),
                               preferred_element_type=f32,
                                precision=lax.Precision.HIGHEST)
  pcnt[...] += jnp.sum(p, axis=0)

  @pl.when(i == N_BLK - 1)
  def _():
    pooled = psum[...] / jnp.maximum(pcnt[...], 1.0)[:, None]
    head_o[...] = (lax.dot_general(pooled, wh[...], dnt,
                                   preferred_element_type=f32,
                                precision=lax.Precision.HIGHEST)
                   + bh[...][None, :])


def _tc_dense(aff0, aff1, abt, cff, cbt, xt, tb3, wpt, wpe,
              wlff, wlbt, wrff, wrbt, blff, blbt, wlin, blin, wh, bh):
  f32 = jnp.float32
  full = lambda shp: pl.BlockSpec(shp, lambda i: tuple(0 for _ in shp))
  grid_spec = pltpu.PrefetchScalarGridSpec(
      num_scalar_prefetch=0,
      grid=(N_BLK,),
      in_specs=[
          pl.BlockSpec((2, R_BLK, 128), lambda i: (0, i, 0)),
          pl.BlockSpec((2, R_BLK, 128), lambda i: (0, i, 0)),
          pl.BlockSpec((2, R_BLK, 128), lambda i: (0, i, 0)),
          pl.BlockSpec((2, R_BLK, 16), lambda i: (0, i, 0)),
          pl.BlockSpec((2, R_BLK, 16), lambda i: (0, i, 0)),
          pl.BlockSpec((R_BLK, 256), lambda i: (i, 0)),
          pl.BlockSpec((1, 1, R_BLK), lambda i: (i, 0, 0)),
          full((H, 256)), full((H, 128)),
          full((H, H)), full((H, H)), full((H, H)), full((H, H)),
          full((H,)), full((H,)),
          full((H, H)), full((H,)),
          full((NUM_GRAPHS, H)), full((NUM_GRAPHS,)),
      ],
      out_specs=[
          pl.BlockSpec((NUM_GRAPHS, NUM_GRAPHS), lambda i: (0, 0)),
      ],
      scratch_shapes=[
          pltpu.VMEM((256, H), f32),
          pltpu.VMEM((128, H), f32),
          pltpu.VMEM((256, H), f32),
          pltpu.VMEM((NUM_GRAPHS, H), f32),
          pltpu.VMEM((NUM_GRAPHS,), f32),
      ],
  )
  return pl.pallas_call(
      _tc_body,
      grid_spec=grid_spec,
      out_shape=(jax.ShapeDtypeStruct((NUM_GRAPHS, NUM_GRAPHS), f32),),
  )(aff0, aff1, abt, cff, cbt, xt, tb3, wpt, wpe, wlff, wlbt, wrff, wrbt,
    blff, blbt, wlin, blin, wh, bh)[0]


def kernel(x_trace, x_event, ei_follows, ei_belongs, ei_contains,
           trace_batch, Wp_trace, Wp_event,
           Wl_ff, bl_ff, Wr_ff, Wl_bt, bl_bt, Wr_bt, Wl_ce, bl_ce, Wr_ce,
           Wlin_trace, blin_trace, Wlin_event, blin_event,
           Wact, bact, Wtime, btime, Wrem, brem):
  i32 = jnp.int32
  f32 = jnp.float32
  zpad = jnp.zeros((PAD_E,), i32)
  dpad = jnp.full((PAD_E,), DUMP, i32)
  src_ff = jnp.concatenate([ei_follows[0].astype(i32), zpad])
  dst_ff = jnp.concatenate([ei_follows[1].astype(i32), dpad]).reshape(
      NW, NCH, K)
  src_bt = jnp.concatenate([ei_belongs[0].astype(i32), zpad]).reshape(
      NW, NCH, K)
  dst_bt = jnp.concatenate([ei_belongs[1].astype(i32), dpad]).reshape(
      NW, NCH, K)
  sf0 = (src_ff * 2).reshape(NW, NCH, K)
  sf1 = (src_ff * 2 + 1).reshape(NW, NCH, K)
  xt2 = x_trace.reshape(2 * N_T, 128)
  zrow = jnp.zeros((N_PAD, 128), f32)
  zcnt = jnp.zeros((N_PAD, 16), f32)
  ones_h = jnp.ones((K, 16), f32)

  aff0, aff1, abt, cff, cbt = _sc_aggregate(
      xt2, x_event, sf0, sf1, src_bt, dst_ff, dst_bt, zrow, zcnt, ones_h)

  tb3 = trace_batch.astype(i32).reshape(N_BLK, 1, R_BLK)
  npad = NUM_GRAPHS - NUM_CLASSES - 2
  wh = jnp.concatenate([Wact, Wtime, Wrem, jnp.zeros((npad, H), f32)], axis=0)
  bh = jnp.concatenate([bact, btime, brem, jnp.zeros((npad,), f32)])
  hout = _tc_dense(
      aff0, aff1, abt, cff, cbt, x_trace, tb3, Wp_trace, Wp_event,
      Wl_ff, Wl_bt, Wr_ff, Wr_bt, bl_ff, bl_bt, Wlin_trace, blin_trace,
      wh, bh)
  return (hout[:, :NUM_CLASSES], hout[:, NUM_CLASSES],
          hout[:, NUM_CLASSES + 1])


# speedup vs baseline: 1.0023x; 1.0023x over previous
import functools

import jax
import jax.numpy as jnp
from jax import lax
from jax.experimental import pallas as pl
from jax.experimental.pallas import tpu as pltpu
from jax.experimental.pallas import tpu_sc as plsc

H = 512
N_T = 10000
N_E = 10000
E = 160000
NUM_GRAPHS = 64
NUM_CLASSES = 32

NW = 32
EPW_P = 5120
K = 64
NCH = EPW_P // K
PAD_E = NW * EPW_P - E
N_PAD = 10240
RPT = N_PAD // 16
DUMP = 10000

R_BLK = 1000
N_BLK = N_T // R_BLK


def _sc_body(xt2, xe, sf0, sf1, sbt, dff, dbt, zrow, zcnt, ones_h,
             off0, off1, obt, ocf, ocb,
             acc, cntf, cntb, sidx, didx, rows0, rows1, ones_v,
             sem0, sem1):
  c = lax.axis_index("c")
  s = lax.axis_index("s")
  wid = s * 2 + c
  base = s * RPT

  def do_pass(x_hbm, src_hbm, dst_hbm, cnt_ref):
    pltpu.sync_copy(src_hbm.at[wid], sidx)
    pltpu.sync_copy(dst_hbm.at[wid], didx)
    dummy = x_hbm.at[pl.ds(0, K)]
    pltpu.async_copy(x_hbm.at[sidx.at[0]], rows0, sem0)

    def scat(buf, j):
      pltpu.sync_copy(buf, acc.at[didx.at[j]], add=True)
      if cnt_ref is not None:
        pltpu.sync_copy(ones_v, cnt_ref.at[didx.at[j]], add=True)

    def pair(g, carry):
      j0 = 2 * g
      pltpu.make_async_copy(dummy, rows0, sem0).wait()
      pltpu.async_copy(x_hbm.at[sidx.at[j0 + 1]], rows1, sem1)
      scat(rows0, j0)
      pltpu.make_async_copy(dummy, rows1, sem1).wait()
      nxt = jnp.minimum(j0 + 2, NCH - 1)
      pltpu.async_copy(x_hbm.at[sidx.at[nxt]], rows0, sem0)
      scat(rows1, j0 + 1)
      return carry

    lax.fori_loop(0, NCH // 2, pair, 0)
    pltpu.make_async_copy(dummy, rows0, sem0).wait()

  def zero_acc():
    pltpu.sync_copy(zrow.at[pl.ds(base, RPT)], acc.at[pl.ds(base, RPT)])

  zero_acc()
  pltpu.sync_copy(zcnt.at[pl.ds(base, RPT)], cntf.at[pl.ds(base, RPT)])
  pltpu.sync_copy(zcnt.at[pl.ds(base, RPT)], cntb.at[pl.ds(base, RPT)])
  pltpu.sync_copy(ones_h, ones_v)
  plsc.subcore_barrier()

  do_pass(xt2, sf0, dff, cntf)
  plsc.subcore_barrier()
  pltpu.sync_copy(acc.at[pl.ds(base, RPT)], off0.at[c, pl.ds(base, RPT)])
  zero_acc()
  plsc.subcore_barrier()

  do_pass(xt2, sf1, dff, None)
  plsc.subcore_barrier()
  pltpu.sync_copy(acc.at[pl.ds(base, RPT)], off1.at[c, pl.ds(base, RPT)])
  zero_acc()
  plsc.subcore_barrier()

  do_pass(xe, sbt, dbt, cntb)
  plsc.subcore_barrier()
  pltpu.sync_copy(acc.at[pl.ds(base, RPT)], obt.at[c, pl.ds(base, RPT)])
  pltpu.sync_copy(cntf.at[pl.ds(base, RPT)], ocf.at[c, pl.ds(base, RPT)])
  pltpu.sync_copy(cntb.at[pl.ds(base, RPT)], ocb.at[c, pl.ds(base, RPT)])


def _sc_aggregate(xt2, xe, sf0, sf1, sbt, dff, dbt, zrow, zcnt, ones_h):
  mesh = plsc.VectorSubcoreMesh(core_axis_name="c", subcore_axis_name="s")
  f32 = jnp.float32
  return pl.kernel(
      _sc_body,
      out_type=(
          jax.ShapeDtypeStruct((2, N_PAD, 128), f32),
          jax.ShapeDtypeStruct((2, N_PAD, 128), f32),
          jax.ShapeDtypeStruct((2, N_PAD, 128), f32),
          jax.ShapeDtypeStruct((2, N_PAD, 16), f32),
          jax.ShapeDtypeStruct((2, N_PAD, 16), f32),
      ),
      mesh=mesh,
      scratch_types=[
          pltpu.VMEM_SHARED((N_PAD, 128), f32),
          pltpu.VMEM_SHARED((N_PAD, 16), f32),
          pltpu.VMEM_SHARED((N_PAD, 16), f32),
          pltpu.VMEM((NCH, K), jnp.int32),
          pltpu.VMEM((NCH, K), jnp.int32),
          pltpu.VMEM((K, 128), f32),
          pltpu.VMEM((K, 128), f32),
          pltpu.VMEM((K, 16), f32),
          pltpu.SemaphoreType.DMA,
          pltpu.SemaphoreType.DMA,
      ],
      compiler_params=pltpu.CompilerParams(use_tc_tiling_on_sc=False),
  )(xt2, xe, sf0, sf1, sbt, dff, dbt, zrow, zcnt, ones_h)


def _tc_body(aff0, aff1, abt, cff, cbt, xt, tb,
             wpt, wpe, wlff, wlbt, wrff, wrbt, blff, blbt,
             wlin, blin, wh, bh,
             head_o,
             A_ff, A_bt, A_r, psum, pcnt):
  i = pl.program_id(0)
  f32 = jnp.float32
  dn = (((0,), (1,)), ((), ()))

  @pl.when(i == 0)
  def _():
    A_ff[...] = lax.dot_general(wpt[...], wlff[...], dn,
                                preferred_element_type=f32,
                                precision=lax.Precision.HIGHEST)
    A_bt[...] = lax.dot_general(wpe[...], wlbt[...], dn,
                                preferred_element_type=f32,
                                precision=lax.Precision.HIGHEST)
    A_r[...] = lax.dot_general(wpt[...], wrff[...] + wrbt[...], dn,
                               preferred_element_type=f32,
                                precision=lax.Precision.HIGHEST)
    psum[...] = jnp.zeros_like(psum)
    pcnt[...] = jnp.zeros_like(pcnt)

  nff = jnp.maximum(cff[0, :, 0] + cff[1, :, 0], 1.0)
  nbt = jnp.maximum(cbt[0, :, 0] + cbt[1, :, 0], 1.0)
  mff = jnp.concatenate([aff0[0] + aff0[1], aff1[0] + aff1[1]], axis=1)
  mff = mff / nff[:, None]
  mbt = (abt[0] + abt[1]) / nbt[:, None]

  dnm = (((1,), (0,)), ((), ()))
  o = (lax.dot_general(mff, A_ff[...], dnm, preferred_element_type=f32,
                                precision=lax.Precision.HIGHEST)
       + lax.dot_general(mbt, A_bt[...], dnm, preferred_element_type=f32,
                                precision=lax.Precision.HIGHEST)
       + lax.dot_general(xt[...], A_r[...], dnm, preferred_element_type=f32,
                                precision=lax.Precision.HIGHEST)
       + (blff[...] + blbt[...])[None, :])
  dnt = (((1,), (1,)), ((), ()))
  t = jnp.maximum(
      lax.dot_general(o, wlin[...], dnt, preferred_element_type=f32,
                                precision=lax.Precision.HIGHEST)
      + blin[...][None, :], 0.0)

  ids = tb[0, 0, :]
  p = (ids[:, None] == lax.broadcasted_iota(jnp.int32, (R_BLK, NUM_GRAPHS),
                                            1)).astype(f32)
  psum[...] += lax.dot_general(p, t, (((0,), (0,)), ((), ())),
                               preferred_element_type=f32,
                                precision=lax.Precision.HIGHEST)
  pcnt[...] += jnp.sum(p, axis=0)

  @pl.when(i == N_BLK - 1)
  def _():
    pooled = psum[...] / jnp.maximum(pcnt[...], 1.0)[:, None]
    head_o[...] = (lax.dot_general(pooled, wh[...], dnt,
                                   preferred_element_type=f32,
                                precision=lax.Precision.HIGHEST)
                   + bh[...][None, :])


def _tc_dense(aff0, aff1, abt, cff, cbt, xt, tb3, wpt, wpe,
              wlff, wlbt, wrff, wrbt, blff, blbt, wlin, blin, wh, bh):
  f32 = jnp.float32
  full = lambda shp: pl.BlockSpec(shp, lambda i: tuple(0 for _ in shp))
  grid_spec = pltpu.PrefetchScalarGridSpec(
      num_scalar_prefetch=0,
      grid=(N_BLK,),
      in_specs=[
          pl.BlockSpec((2, R_BLK, 128), lambda i: (0, i, 0)),
          pl.BlockSpec((2, R_BLK, 128), lambda i: (0, i, 0)),
          pl.BlockSpec((2, R_BLK, 128), lambda i: (0, i, 0)),
          pl.BlockSpec((2, R_BLK, 16), lambda i: (0, i, 0)),
          pl.BlockSpec((2, R_BLK, 16), lambda i: (0, i, 0)),
          pl.BlockSpec((R_BLK, 256), lambda i: (i, 0)),
          pl.BlockSpec((1, 1, R_BLK), lambda i: (i, 0, 0)),
          full((H, 256)), full((H, 128)),
          full((H, H)), full((H, H)), full((H, H)), full((H, H)),
          full((H,)), full((H,)),
          full((H, H)), full((H,)),
          full((NUM_GRAPHS, H)), full((NUM_GRAPHS,)),
      ],
      out_specs=[
          pl.BlockSpec((NUM_GRAPHS, NUM_GRAPHS), lambda i: (0, 0)),
      ],
      scratch_shapes=[
          pltpu.VMEM((256, H), f32),
          pltpu.VMEM((128, H), f32),
          pltpu.VMEM((256, H), f32),
          pltpu.VMEM((NUM_GRAPHS, H), f32),
          pltpu.VMEM((NUM_GRAPHS,), f32),
      ],
  )
  return pl.pallas_call(
      _tc_body,
      grid_spec=grid_spec,
      out_shape=(jax.ShapeDtypeStruct((NUM_GRAPHS, NUM_GRAPHS), f32),),
  )(aff0, aff1, abt, cff, cbt, xt, tb3, wpt, wpe, wlff, wlbt, wrff, wrbt,
    blff, blbt, wlin, blin, wh, bh)[0]


def kernel(x_trace, x_event, ei_follows, ei_belongs, ei_contains,
           trace_batch, Wp_trace, Wp_event,
           Wl_ff, bl_ff, Wr_ff, Wl_bt, bl_bt, Wr_bt, Wl_ce, bl_ce, Wr_ce,
           Wlin_trace, blin_trace, Wlin_event, blin_event,
           Wact, bact, Wtime, btime, Wrem, brem):
  i32 = jnp.int32
  f32 = jnp.float32
  zpad = jnp.zeros((PAD_E,), i32)
  dpad = DUMP + jnp.arange(PAD_E, dtype=i32) % (N_PAD - DUMP)
  src_ff = jnp.concatenate([ei_follows[0].astype(i32), zpad])
  dst_ff = jnp.concatenate([ei_follows[1].astype(i32), dpad]).reshape(
      NW, NCH, K)
  src_bt = jnp.concatenate([ei_belongs[0].astype(i32), zpad]).reshape(
      NW, NCH, K)
  dst_bt = jnp.concatenate([ei_belongs[1].astype(i32), dpad]).reshape(
      NW, NCH, K)
  sf0 = (src_ff * 2).reshape(NW, NCH, K)
  sf1 = (src_ff * 2 + 1).reshape(NW, NCH, K)
  xt2 = x_trace.reshape(2 * N_T, 128)
  zrow = jnp.zeros((N_PAD, 128), f32)
  zcnt = jnp.zeros((N_PAD, 16), f32)
  ones_h = jnp.ones((K, 16), f32)

  aff0, aff1, abt, cff, cbt = _sc_aggregate(
      xt2, x_event, sf0, sf1, src_bt, dst_ff, dst_bt, zrow, zcnt, ones_h)

  tb3 = trace_batch.astype(i32).reshape(N_BLK, 1, R_BLK)
  npad = NUM_GRAPHS - NUM_CLASSES - 2
  wh = jnp.concatenate([Wact, Wtime, Wrem, jnp.zeros((npad, H), f32)], axis=0)
  bh = jnp.concatenate([bact, btime, brem, jnp.zeros((npad,), f32)])
  hout = _tc_dense(
      aff0, aff1, abt, cff, cbt, x_trace, tb3, Wp_trace, Wp_event,
      Wl_ff, Wl_bt, Wr_ff, Wr_bt, bl_ff, bl_bt, Wlin_trace, blin_trace,
      wh, bh)
  return (hout[:, :NUM_CLASSES], hout[:, NUM_CLASSES],
          hout[:, NUM_CLASSES + 1])
